# baseline (speedup 1.0000x reference)
import functools

import jax
import jax.numpy as jnp
from jax.experimental import pallas as pl
from jax.experimental.pallas import tpu as pltpu

K = 8192
D = 32
TN = 256


def _vq_kernel(x_ref, embt_ref, emb_ref, zq_st_ref, zq_ref, idx_ref,
               usage_ref, stats_ref, *, n_steps):
    i = pl.program_id(0)
    x = x_ref[...]
    norm = jnp.sqrt(jnp.sum(x * x, axis=1, keepdims=True))
    xn = x / jnp.maximum(norm, 1e-8)
    dots = jnp.dot(xn, embt_ref[...], preferred_element_type=jnp.float32)
    iota = jax.lax.broadcasted_iota(jnp.int32, (TN, K), 1)
    row_max = jnp.max(dots, axis=1, keepdims=True)
    idx = jnp.min(jnp.where(dots == row_max, iota, K), axis=1)
    idx_ref[...] = idx
    onehot = (iota == idx[:, None]).astype(jnp.float32)
    zq = jnp.dot(onehot, emb_ref[...], preferred_element_type=jnp.float32)
    zq_ref[...] = zq
    zq_st_ref[...] = x + (zq - x)
    counts = jnp.sum(onehot, axis=0)

    @pl.when(i == 0)
    def _init():
        usage_ref[...] = counts

    @pl.when(i > 0)
    def _acc():
        usage_ref[...] = usage_ref[...] + counts

    @pl.when(i == n_steps - 1)
    def _stats():
        usage = usage_ref[...]
        total = jnp.sum(usage)
        probs = usage / jnp.maximum(total, 1.0)
        safe = jnp.where(probs > 0, probs, 1.0)
        perp = jnp.exp(-jnp.sum(probs * jnp.log(safe)))
        dead = jnp.mean((usage == 0).astype(jnp.float32))
        sel = jax.lax.broadcasted_iota(jnp.int32, (2,), 0) == 0
        stats_ref[...] = jnp.where(sel, perp, dead)


def kernel(z_e, embedding):
    B, L, Dv = z_e.shape
    N = B * L
    n_steps = N // TN
    flat = z_e.reshape(N, Dv)
    embt = embedding.T

    grid_spec = pl.GridSpec(
        grid=(n_steps,),
        in_specs=[
            pl.BlockSpec((TN, Dv), lambda i: (i, 0)),
            pl.BlockSpec((Dv, K), lambda i: (0, 0)),
            pl.BlockSpec((K, Dv), lambda i: (0, 0)),
        ],
        out_specs=[
            pl.BlockSpec((TN, Dv), lambda i: (i, 0)),
            pl.BlockSpec((TN, Dv), lambda i: (i, 0)),
            pl.BlockSpec((TN,), lambda i: (i,)),
            pl.BlockSpec((K,), lambda i: (0,)),
            pl.BlockSpec((2,), lambda i: (0,)),
        ],
    )
    out_shape = [
        jax.ShapeDtypeStruct((N, Dv), jnp.float32),
        jax.ShapeDtypeStruct((N, Dv), jnp.float32),
        jax.ShapeDtypeStruct((N,), jnp.int32),
        jax.ShapeDtypeStruct((K,), jnp.float32),
        jax.ShapeDtypeStruct((2,), jnp.float32),
    ]
    zq_st, zq, idx, usage, stats = pl.pallas_call(
        functools.partial(_vq_kernel, n_steps=n_steps),
        grid_spec=grid_spec,
        out_shape=out_shape,
    )(flat, embt, embedding)
    return (zq_st.reshape(B, L, Dv), zq.reshape(B, L, Dv),
            idx.reshape(B, L), stats)


# speedup vs baseline: 1.4387x; 1.4387x over previous
import functools

import jax
import jax.numpy as jnp
from jax.experimental import pallas as pl
from jax.experimental.pallas import tpu as pltpu
from jax.experimental.pallas import tpu_sc as plsc

K = 8192
D = 32
TN = 256
GW = 128


def _argmax_kernel(x_ref, embt_ref, idx_ref, usage_ref, stats_ref, *, n_steps):
    i = pl.program_id(0)
    x = x_ref[...]
    norm = jnp.sqrt(jnp.sum(x * x, axis=1, keepdims=True))
    xn = x / jnp.maximum(norm, 1e-8)
    dots = jnp.dot(xn, embt_ref[...], preferred_element_type=jnp.float32)
    iota = jax.lax.broadcasted_iota(jnp.int32, (TN, K), 1)
    row_max = jnp.max(dots, axis=1, keepdims=True)
    mask = dots == row_max
    idx_ref[...] = jnp.min(jnp.where(mask, iota, K), axis=1)
    counts = jnp.sum(mask.astype(jnp.float32), axis=0)

    @pl.when(i == 0)
    def _init():
        usage_ref[...] = counts

    @pl.when(i > 0)
    def _acc():
        usage_ref[...] = usage_ref[...] + counts

    @pl.when(i == n_steps - 1)
    def _stats():
        usage = usage_ref[...]
        total = jnp.sum(usage)
        probs = usage / jnp.maximum(total, 1.0)
        safe = jnp.where(probs > 0, probs, 1.0)
        perp = jnp.exp(-jnp.sum(probs * jnp.log(safe)))
        dead = jnp.mean((usage == 0).astype(jnp.float32))
        sel = jax.lax.broadcasted_iota(jnp.int32, (2,), 0) == 0
        stats_ref[...] = jnp.where(sel, perp, dead)


NC = 2
NS = 16
NW = NC * NS


DP = 128


def _sc_gather(emb_pad, idx, n):
    b_per_w = n // NW
    half = b_per_w // 2
    n_chunks = half // GW
    mesh = plsc.VectorSubcoreMesh(core_axis_name="c", subcore_axis_name="s")

    @functools.partial(
        pl.kernel,
        out_type=jax.ShapeDtypeStruct((n, DP), jnp.float32),
        mesh=mesh,
        scratch_types=[
            pltpu.VMEM((b_per_w,), jnp.int32),
            pltpu.VMEM((half, DP), jnp.float32),
            pltpu.SemaphoreType.DMA,
        ],
    )
    def gather_kernel(emb_hbm, idx_hbm, out_hbm, idx_v, rows_v, sem):
        wid = jax.lax.axis_index("s") * NC + jax.lax.axis_index("c")
        base = wid * b_per_w
        pltpu.sync_copy(idx_hbm.at[pl.ds(base, b_per_w)], idx_v)
        for h in range(2):
            copies = [
                pltpu.async_copy(
                    emb_hbm.at[idx_v.at[pl.ds(h * half + c * GW, GW)]],
                    rows_v.at[pl.ds(c * GW, GW), :],
                    sem,
                )
                for c in range(n_chunks)
            ]
            for cp in copies:
                cp.wait()
            pltpu.sync_copy(rows_v, out_hbm.at[pl.ds(base + h * half, half)])

    return gather_kernel(emb_pad, idx)


def kernel(z_e, embedding):
    B, L, Dv = z_e.shape
    N = B * L
    n_steps = N // TN
    flat = z_e.reshape(N, Dv)
    embt = embedding.T

    grid_spec = pl.GridSpec(
        grid=(n_steps,),
        in_specs=[
            pl.BlockSpec((TN, Dv), lambda i: (i, 0)),
            pl.BlockSpec((Dv, K), lambda i: (0, 0)),
        ],
        out_specs=[
            pl.BlockSpec((TN,), lambda i: (i,)),
            pl.BlockSpec((K,), lambda i: (0,)),
            pl.BlockSpec((2,), lambda i: (0,)),
        ],
    )
    out_shape = [
        jax.ShapeDtypeStruct((N,), jnp.int32),
        jax.ShapeDtypeStruct((K,), jnp.float32),
        jax.ShapeDtypeStruct((2,), jnp.float32),
    ]
    idx, usage, stats = pl.pallas_call(
        functools.partial(_argmax_kernel, n_steps=n_steps),
        grid_spec=grid_spec,
        out_shape=out_shape,
    )(flat, embt)

    emb_pad = jnp.pad(embedding, ((0, 0), (0, DP - Dv)))
    zq = _sc_gather(emb_pad, idx, N)[:, :Dv].reshape(B, L, Dv)
    return (zq, zq, idx.reshape(B, L), stats)


# speedup vs baseline: 2.6840x; 1.8655x over previous
import functools

import jax
import jax.numpy as jnp
from jax.experimental import pallas as pl
from jax.experimental.pallas import tpu as pltpu
from jax.experimental.pallas import tpu_sc as plsc

K = 8192
D = 32
TN = 256
TKL = 512
GW = 128
NC = 2
NS = 16
NW = NC * NS
DP = 128


def _argmax_kernel(x_ref, embt_ref, idx_ref):
    x = x_ref[...]
    norm = jnp.sqrt(jnp.sum(x * x, axis=1, keepdims=True))
    xn = x / jnp.maximum(norm, 1e-8)
    dots = jnp.dot(xn, embt_ref[...], preferred_element_type=jnp.float32)
    n_tiles = K // TKL
    best_v = dots[:, :TKL]
    best_t = jnp.zeros((TN, TKL), jnp.int32)
    for t in range(1, n_tiles):
        d = dots[:, t * TKL:(t + 1) * TKL]
        gt = d > best_v
        best_v = jnp.where(gt, d, best_v)
        best_t = jnp.where(gt, t, best_t)
    lane = jax.lax.broadcasted_iota(jnp.int32, (TN, TKL), 1)
    full_idx = best_t * TKL + lane
    row_max = jnp.max(best_v, axis=1, keepdims=True)
    cand = jnp.where(best_v == row_max, full_idx, K)
    idx_ref[...] = jnp.min(cand, axis=1)


def _sc_gather_hist(emb_pad, idx, zeros_k, ones_w, n):
    b_per_w = n // NW
    half = b_per_w // 2
    n_chunks = half // GW
    mesh = plsc.VectorSubcoreMesh(core_axis_name="c", subcore_axis_name="s")

    @functools.partial(
        pl.kernel,
        out_type=[
            jax.ShapeDtypeStruct((n, DP), jnp.float32),
            jax.ShapeDtypeStruct((NC, K), jnp.float32),
        ],
        mesh=mesh,
        scratch_types=[
            pltpu.VMEM((b_per_w,), jnp.int32),
            pltpu.VMEM((half, DP), jnp.float32),
            pltpu.VMEM((b_per_w,), jnp.float32),
            pltpu.VMEM_SHARED((K,), jnp.float32),
            pltpu.SemaphoreType.DMA,
        ],
    )
    def gather_kernel(emb_hbm, idx_hbm, zeros_hbm, ones_hbm, out_hbm,
                      usage_hbm, idx_v, rows_v, ones_v, usage_sh, sem):
        cid = jax.lax.axis_index("c")
        sid = jax.lax.axis_index("s")
        wid = sid * NC + cid
        base = wid * b_per_w
        pltpu.sync_copy(idx_hbm.at[pl.ds(base, b_per_w)], idx_v)
        pltpu.sync_copy(ones_hbm, ones_v)

        @pl.when(sid == 0)
        def _zero():
            pltpu.sync_copy(zeros_hbm, usage_sh)

        plsc.subcore_barrier()
        for c in range(b_per_w // GW):
            pltpu.sync_copy(
                ones_v.at[pl.ds(c * GW, GW)],
                usage_sh.at[idx_v.at[pl.ds(c * GW, GW)]],
                add=True,
            )
        for h in range(2):
            copies = [
                pltpu.async_copy(
                    emb_hbm.at[idx_v.at[pl.ds(h * half + c * GW, GW)]],
                    rows_v.at[pl.ds(c * GW, GW), :],
                    sem,
                )
                for c in range(n_chunks)
            ]
            for cp in copies:
                cp.wait()
            pltpu.sync_copy(rows_v, out_hbm.at[pl.ds(base + h * half, half)])

        plsc.subcore_barrier()

        @pl.when(sid == 0)
        def _write_usage():
            pltpu.sync_copy(usage_sh, usage_hbm.at[cid])

    return gather_kernel(emb_pad, idx, zeros_k, ones_w)


def _stats_kernel(usage2_ref, stats_ref):
    usage = usage2_ref[0, :] + usage2_ref[1, :]
    total = jnp.sum(usage)
    probs = usage / jnp.maximum(total, 1.0)
    safe = jnp.where(probs > 0, probs, 1.0)
    perp = jnp.exp(-jnp.sum(probs * jnp.log(safe)))
    dead = jnp.mean((usage == 0).astype(jnp.float32))
    sel = jax.lax.broadcasted_iota(jnp.int32, (2,), 0) == 0
    stats_ref[...] = jnp.where(sel, perp, dead)


def kernel(z_e, embedding):
    B, L, Dv = z_e.shape
    N = B * L
    n_steps = N // TN
    flat = z_e.reshape(N, Dv)
    embt = embedding.T

    idx = pl.pallas_call(
        _argmax_kernel,
        grid_spec=pl.GridSpec(
            grid=(n_steps,),
            in_specs=[
                pl.BlockSpec((TN, Dv), lambda i: (i, 0)),
                pl.BlockSpec((Dv, K), lambda i: (0, 0)),
            ],
            out_specs=pl.BlockSpec((TN,), lambda i: (i,)),
        ),
        out_shape=jax.ShapeDtypeStruct((N,), jnp.int32),
    )(flat, embt)

    emb_pad = jnp.pad(embedding, ((0, 0), (0, DP - Dv)))
    zeros_k = jnp.zeros((K,), jnp.float32)
    ones_w = jnp.ones((N // NW,), jnp.float32)
    zq_pad, usage2 = _sc_gather_hist(emb_pad, idx, zeros_k, ones_w, N)
    zq = zq_pad[:, :Dv].reshape(B, L, Dv)

    stats = pl.pallas_call(
        _stats_kernel,
        out_shape=jax.ShapeDtypeStruct((2,), jnp.float32),
    )(usage2)

    return (zq, zq, idx.reshape(B, L), stats)


# speedup vs baseline: 2.9471x; 1.0980x over previous
import functools

import jax
import jax.numpy as jnp
from jax.experimental import pallas as pl
from jax.experimental.pallas import tpu as pltpu
from jax.experimental.pallas import tpu_sc as plsc

K = 8192
D = 32
TN = 512
TKL = 256
GW = 128
NC = 2
NS = 16
NW = NC * NS
DP = 128


def _argmax_kernel(x_ref, embt_ref, idx_ref):
    x = x_ref[...]
    norm = jnp.sqrt(jnp.sum(x * x, axis=1, keepdims=True))
    xn = x / jnp.maximum(norm, 1e-8)
    dots = jnp.dot(xn, embt_ref[...], preferred_element_type=jnp.float32)
    n_tiles = K // TKL
    best_v = dots[:, :TKL]
    best_t = jnp.zeros((TN, TKL), jnp.int32)
    for t in range(1, n_tiles):
        d = dots[:, t * TKL:(t + 1) * TKL]
        gt = d > best_v
        best_v = jnp.where(gt, d, best_v)
        best_t = jnp.where(gt, t, best_t)
    lane = jax.lax.broadcasted_iota(jnp.int32, (TN, TKL), 1)
    full_idx = best_t * TKL + lane
    row_max = jnp.max(best_v, axis=1, keepdims=True)
    cand = jnp.where(best_v == row_max, full_idx, K)
    idx_ref[...] = jnp.min(cand, axis=1)


def _sc_gather_hist(emb_pad, idx, zeros_k, ones_w, n):
    b_per_w = n // NW
    half = b_per_w // 2
    n_chunks = half // GW
    mesh = plsc.VectorSubcoreMesh(core_axis_name="c", subcore_axis_name="s")

    @functools.partial(
        pl.kernel,
        out_type=[
            jax.ShapeDtypeStruct((n, DP), jnp.float32),
            jax.ShapeDtypeStruct((NC, K), jnp.float32),
        ],
        mesh=mesh,
        scratch_types=[
            pltpu.VMEM((b_per_w,), jnp.int32),
            pltpu.VMEM((half, DP), jnp.float32),
            pltpu.VMEM((b_per_w,), jnp.float32),
            pltpu.VMEM_SHARED((K,), jnp.float32),
            pltpu.SemaphoreType.DMA,
        ],
    )
    def gather_kernel(emb_hbm, idx_hbm, zeros_hbm, ones_hbm, out_hbm,
                      usage_hbm, idx_v, rows_v, ones_v, usage_sh, sem):
        cid = jax.lax.axis_index("c")
        sid = jax.lax.axis_index("s")
        wid = sid * NC + cid
        base = wid * b_per_w
        pltpu.sync_copy(idx_hbm.at[pl.ds(base, b_per_w)], idx_v)
        pltpu.sync_copy(ones_hbm, ones_v)

        @pl.when(sid == 0)
        def _zero():
            pltpu.sync_copy(zeros_hbm, usage_sh)

        plsc.subcore_barrier()
        for c in range(b_per_w // GW):
            pltpu.sync_copy(
                ones_v.at[pl.ds(c * GW, GW)],
                usage_sh.at[idx_v.at[pl.ds(c * GW, GW)]],
                add=True,
            )
        for h in range(2):
            copies = [
                pltpu.async_copy(
                    emb_hbm.at[idx_v.at[pl.ds(h * half + c * GW, GW)]],
                    rows_v.at[pl.ds(c * GW, GW), :],
                    sem,
                )
                for c in range(n_chunks)
            ]
            for cp in copies:
                cp.wait()
            pltpu.sync_copy(rows_v, out_hbm.at[pl.ds(base + h * half, half)])

        plsc.subcore_barrier()

        @pl.when(sid == 0)
        def _write_usage():
            pltpu.sync_copy(usage_sh, usage_hbm.at[cid])

    return gather_kernel(emb_pad, idx, zeros_k, ones_w)


def _stats_kernel(usage2_ref, stats_ref):
    usage = usage2_ref[0, :] + usage2_ref[1, :]
    total = jnp.sum(usage)
    probs = usage / jnp.maximum(total, 1.0)
    safe = jnp.where(probs > 0, probs, 1.0)
    perp = jnp.exp(-jnp.sum(probs * jnp.log(safe)))
    dead = jnp.mean((usage == 0).astype(jnp.float32))
    sel = jax.lax.broadcasted_iota(jnp.int32, (2,), 0) == 0
    stats_ref[...] = jnp.where(sel, perp, dead)


def kernel(z_e, embedding):
    B, L, Dv = z_e.shape
    N = B * L
    n_steps = N // TN
    flat = z_e.reshape(N, Dv)
    embt = embedding.T

    idx = pl.pallas_call(
        _argmax_kernel,
        grid_spec=pl.GridSpec(
            grid=(n_steps,),
            in_specs=[
                pl.BlockSpec((TN, Dv), lambda i: (i, 0)),
                pl.BlockSpec((Dv, K), lambda i: (0, 0)),
            ],
            out_specs=pl.BlockSpec((TN,), lambda i: (i,)),
        ),
        out_shape=jax.ShapeDtypeStruct((N,), jnp.int32),
    )(flat, embt)

    emb_pad = jnp.pad(embedding, ((0, 0), (0, DP - Dv)))
    zeros_k = jnp.zeros((K,), jnp.float32)
    ones_w = jnp.ones((N // NW,), jnp.float32)
    zq_pad, usage2 = _sc_gather_hist(emb_pad, idx, zeros_k, ones_w, N)
    zq = zq_pad[:, :Dv].reshape(B, L, Dv)

    stats = pl.pallas_call(
        _stats_kernel,
        out_shape=jax.ShapeDtypeStruct((2,), jnp.float32),
    )(usage2)

    return (zq, zq, idx.reshape(B, L), stats)
